# baseline (speedup 1.0000x reference)
import functools

import jax
import jax.numpy as jnp
from jax import lax
from jax.experimental import pallas as pl
from jax.experimental.pallas import tpu as pltpu
from jax.experimental.pallas import tpu_sc as plsc

_N = 10000
_E = 320000
_H = 128
_NG = 8

_NC = 2
_NS = 16
_NW = _NC * _NS
_CHUNK = 128
_EPT = 10240
_NCHUNKS = _EPT // _CHUNK
_EPAD = _EPT * _NW
_NPAD = 10240
_RPT = _NPAD // _NS

_K0 = 160
_K1 = 0

_BLK = 400
_NBLK = _N // _BLK

def _mesh():
  return plsc.VectorSubcoreMesh(core_axis_name="c", subcore_axis_name="s",
                                num_cores=_NC, num_subcores=_NS)


def _sc_degree(idx2, ones128, zeros128):
  @functools.partial(
      pl.kernel,
      mesh=_mesh(),
      out_type=jax.ShapeDtypeStruct((_NC * _NPAD, _H), jnp.float32),
      scratch_types=[
          pltpu.VMEM((2, _CHUNK), jnp.int32),
          pltpu.VMEM((2, _CHUNK), jnp.int32),
          pltpu.VMEM((_CHUNK, _H), jnp.float32),
          pltpu.VMEM_SHARED((_NPAD, _H), jnp.float32),
          pltpu.SemaphoreType.DMA,
          pltpu.SemaphoreType.DMA,
      ],
  )
  def k(idx_hbm, ones_hbm, zeros_hbm, out_hbm, i0, i1, ones_v, acc, s0, s1):
    c = lax.axis_index("c")
    s = lax.axis_index("s")
    pltpu.sync_copy(zeros_hbm, acc.at[pl.ds(s * _RPT, _RPT)])
    pltpu.sync_copy(ones_hbm, ones_v)
    plsc.subcore_barrier()
    bch = (c * _NS + s) * _NCHUNKS
    ii = (i0, i1)
    ss = (s0, s1)

    pltpu.async_copy(idx_hbm.at[bch], i0, s0).wait()
    pltpu.async_copy(idx_hbm.at[bch + 1], i1, s1)

    @pl.loop(0, _NCHUNKS // 2)
    def _(p):
      for b in range(2):
        ch = 2 * p + b
        ib, inx = ii[b], ii[1 - b]
        pltpu.sync_copy(ones_v, acc.at[ib.at[1]], add=True)

        @pl.when(ch + 2 < _NCHUNKS)
        def _():
          pltpu.async_copy(idx_hbm.at[bch + ch + 2], ib, ss[b])

        @pl.when(ch + 1 < _NCHUNKS)
        def _():
          pltpu.make_async_copy(idx_hbm.at[bch], inx, ss[1 - b]).wait()

    plsc.subcore_barrier()
    row0 = c * _NPAD + s * _RPT
    pltpu.sync_copy(acc.at[pl.ds(s * _RPT, _RPT)], out_hbm.at[pl.ds(row0, _RPT)])

  return k(idx2, ones128, zeros128)


def _sc_scatter(g, idx2, src_pad, dst_pad, zeros128):
  @functools.partial(
      pl.kernel,
      mesh=_mesh(),
      out_type=jax.ShapeDtypeStruct((_NC * _NPAD, _H), jnp.float32),
      scratch_types=[
          pltpu.VMEM((2, _CHUNK), jnp.int32),
          pltpu.VMEM((2, _CHUNK), jnp.int32),
          pltpu.VMEM((_CHUNK,), jnp.int32),
          pltpu.VMEM((_CHUNK,), jnp.int32),
          pltpu.VMEM((_CHUNK, _H), jnp.float32),
          pltpu.VMEM((_CHUNK, _H), jnp.float32),
          pltpu.VMEM_SHARED((_NPAD, _H), jnp.float32),
          pltpu.SemaphoreType.DMA,
          pltpu.SemaphoreType.DMA,
          pltpu.SemaphoreType.DMA,
          pltpu.SemaphoreType.DMA,
      ],
  )
  def k(g_hbm, idx_hbm, src_hbm, dst_hbm, zeros_hbm, out_hbm,
        i0, i1, sidx, didx, r0, r1, acc, si0, si1, sg0, sg1):
    c = lax.axis_index("c")
    s = lax.axis_index("s")
    pltpu.sync_copy(zeros_hbm, acc.at[pl.ds(s * _RPT, _RPT)])
    plsc.subcore_barrier()

    @pl.when(c == 0)
    def _():
      bch = s * _K0
      ii = (i0, i1)
      rr = (r0, r1)
      si = (si0, si1)
      sg = (sg0, sg1)
      pltpu.async_copy(idx_hbm.at[bch], i0, si0).wait()
      pltpu.async_copy(idx_hbm.at[bch + 1], i1, si1)
      pltpu.async_copy(g_hbm.at[i0.at[0]], r0, sg0)

      @pl.loop(0, (_K0 - 2) // 2)
      def _(p):
        ch0 = 2 * p
        for b in range(2):
          ib, inx = ii[b], ii[1 - b]
          rn, rx = rr[b], rr[1 - b]
          pltpu.make_async_copy(idx_hbm.at[bch], inx, si[1 - b]).wait()
          pltpu.async_copy(g_hbm.at[inx.at[0]], rx, sg[1 - b])
          pltpu.make_async_copy(g_hbm.at[ib.at[0]], rn, sg[b]).wait()
          pltpu.sync_copy(rn, acc.at[ib.at[1]], add=True)
          pltpu.async_copy(idx_hbm.at[bch + ch0 + b + 2], ib, si[b])

      pltpu.make_async_copy(idx_hbm.at[bch], i1, si1).wait()
      pltpu.async_copy(g_hbm.at[i1.at[0]], r1, sg1)
      pltpu.make_async_copy(g_hbm.at[i0.at[0]], r0, sg0).wait()
      pltpu.sync_copy(r0, acc.at[i0.at[1]], add=True)
      pltpu.make_async_copy(g_hbm.at[i1.at[0]], r1, sg1).wait()
      pltpu.sync_copy(r1, acc.at[i1.at[1]], add=True)

    if _K1:
      @pl.when(c == 1)
      def _():
        base = (_NS * _K0 + s * _K1) * _CHUNK

        @pl.loop(0, _K1)
        def _(ch):
          off = base + ch * _CHUNK
          pltpu.sync_copy(src_hbm.at[pl.ds(off, _CHUNK)], sidx)
          pltpu.sync_copy(dst_hbm.at[pl.ds(off, _CHUNK)], didx)
          pltpu.async_copy(g_hbm.at[sidx], r0, sg0).wait()
          pltpu.sync_copy(r0, acc.at[didx], add=True)

    plsc.subcore_barrier()
    row0 = c * _NPAD + s * _RPT
    pltpu.sync_copy(acc.at[pl.ds(s * _RPT, _RPT)], out_hbm.at[pl.ds(row0, _RPT)])

  return k(g, idx2, src_pad, dst_pad, zeros128)


def _mm_body(x_ref, w_ref, o_ref):
  o_ref[...] = jnp.dot(x_ref[...], w_ref[...], preferred_element_type=jnp.float32)


def _matmul(x, w):
  return pl.pallas_call(
      _mm_body,
      grid=(_NBLK,),
      in_specs=[
          pl.BlockSpec((_BLK, x.shape[1]), lambda i: (i, 0)),
          pl.BlockSpec(w.shape, lambda i: (0, 0)),
      ],
      out_specs=pl.BlockSpec((_BLK, w.shape[1]), lambda i: (i, 0)),
      out_shape=jax.ShapeDtypeStruct((_N, w.shape[1]), jnp.float32),
      compiler_params=pltpu.CompilerParams(dimension_semantics=("parallel",)),
  )(x, w)


def _post1_body(h_ref, p0_ref, p1_ref, dinv_ref, g_ref):
  deg = p0_ref[...] + p1_ref[...] + 1.0
  dinv = lax.rsqrt(deg)
  dinv_ref[...] = dinv
  g_ref[...] = h_ref[...] * dinv


def _post1(h1, p0, p1):
  return pl.pallas_call(
      _post1_body,
      grid=(_NBLK,),
      in_specs=[
          pl.BlockSpec((_BLK, _H), lambda i: (i, 0)),
          pl.BlockSpec((_BLK, 1), lambda i: (i, 0)),
          pl.BlockSpec((_BLK, 1), lambda i: (i, 0)),
      ],
      out_specs=[
          pl.BlockSpec((_BLK, 1), lambda i: (i, 0)),
          pl.BlockSpec((_BLK, _H), lambda i: (i, 0)),
      ],
      out_shape=[
          jax.ShapeDtypeStruct((_N, 1), jnp.float32),
          jax.ShapeDtypeStruct((_N, _H), jnp.float32),
      ],
      compiler_params=pltpu.CompilerParams(dimension_semantics=("parallel",)),
  )(h1, p0, p1)


def _layer_body(p0_ref, p1_ref, g_ref, dinv_ref, b_ref, w_ref, o_ref):
  dinv = dinv_ref[...]
  t = dinv * (p0_ref[...] + p1_ref[...] + g_ref[...]) + b_ref[...]
  z = jnp.maximum(t, 0.0)
  o_ref[...] = dinv * jnp.dot(z, w_ref[...], preferred_element_type=jnp.float32)


def _layer(p0, p1, g, dinv, b, w):
  return pl.pallas_call(
      _layer_body,
      grid=(_NBLK,),
      in_specs=[
          pl.BlockSpec((_BLK, _H), lambda i: (i, 0)),
          pl.BlockSpec((_BLK, _H), lambda i: (i, 0)),
          pl.BlockSpec((_BLK, _H), lambda i: (i, 0)),
          pl.BlockSpec((_BLK, 1), lambda i: (i, 0)),
          pl.BlockSpec((1, _H), lambda i: (0, 0)),
          pl.BlockSpec((_H, _H), lambda i: (0, 0)),
      ],
      out_specs=pl.BlockSpec((_BLK, _H), lambda i: (i, 0)),
      out_shape=jax.ShapeDtypeStruct((_N, _H), jnp.float32),
      compiler_params=pltpu.CompilerParams(dimension_semantics=("parallel",)),
  )(p0, p1, g, dinv, b, w)


def _final_body(p0_ref, p1_ref, g_ref, dinv_ref, b3_ref, wl1_ref, bl1_ref,
                wl2_ref, bl2_ref, wl3_ref, bl3_ref, batch_ref, out_ref,
                cnt_ref):
  i = pl.program_id(0)

  @pl.when(i == 0)
  def _():
    out_ref[...] = jnp.zeros_like(out_ref)
    cnt_ref[...] = jnp.zeros_like(cnt_ref)

  dinv = dinv_ref[...]
  t = dinv * (p0_ref[...] + p1_ref[...] + g_ref[...]) + b3_ref[...]
  h1m = jnp.maximum(
      jnp.dot(t, wl1_ref[...], preferred_element_type=jnp.float32)
      + bl1_ref[...], 0.0)
  hm = jnp.maximum(
      jnp.dot(h1m, wl2_ref[...], preferred_element_type=jnp.float32)
      + bl2_ref[...] + t, 0.0)
  hf = jnp.maximum(
      jnp.dot(hm, wl3_ref[...], preferred_element_type=jnp.float32)
      + bl3_ref[...], 0.0)
  onehot = (batch_ref[...] ==
            lax.broadcasted_iota(jnp.int32, (1, _NG), 1)).astype(jnp.float32)
  out_ref[...] += lax.dot_general(
      onehot, hf, (((0,), (0,)), ((), ())), preferred_element_type=jnp.float32)
  cnt_ref[...] += jnp.broadcast_to(jnp.sum(onehot, axis=0)[:, None],
                                   (_NG, _H))

  @pl.when(i == _NBLK - 1)
  def _():
    out_ref[...] = out_ref[...] / jnp.maximum(cnt_ref[...], 1.0)


def _final(p0, p1, g, dinv, b3, wl1, bl1, wl2, bl2, wl3, bl3, batch2d):
  row = lambda i: (i, 0)
  fixed = lambda i: (0, 0)
  return pl.pallas_call(
      _final_body,
      grid=(_NBLK,),
      in_specs=[
          pl.BlockSpec((_BLK, _H), row),
          pl.BlockSpec((_BLK, _H), row),
          pl.BlockSpec((_BLK, _H), row),
          pl.BlockSpec((_BLK, 1), row),
          pl.BlockSpec((1, _H), fixed),
          pl.BlockSpec((_H, _H), fixed),
          pl.BlockSpec((1, _H), fixed),
          pl.BlockSpec((_H, _H), fixed),
          pl.BlockSpec((1, _H), fixed),
          pl.BlockSpec((_H, _H), fixed),
          pl.BlockSpec((1, _H), fixed),
          pl.BlockSpec((_BLK, 1), row),
      ],
      out_specs=pl.BlockSpec((_NG, _H), fixed),
      out_shape=jax.ShapeDtypeStruct((_NG, _H), jnp.float32),
      scratch_shapes=[pltpu.VMEM((_NG, _H), jnp.float32)],
      compiler_params=pltpu.CompilerParams(dimension_semantics=("arbitrary",)),
  )(p0, p1, g, dinv, b3, wl1, bl1, wl2, bl2, wl3, bl3, batch2d)


def kernel(x, edge_index, batch, pos, W1, b1, W2, b2, W3, b3,
           Wl1, bl1, Wl2, bl2, Wl3, bl3):
  del pos
  src = edge_index[0]
  dst = edge_index[1]
  npad = _EPAD - _E
  src_pad = jnp.concatenate([src, jnp.zeros((npad,), jnp.int32)])
  dst_pad = jnp.concatenate([dst, jnp.full((npad,), _N, jnp.int32)])

  idx2 = jnp.stack([src_pad.reshape(-1, _CHUNK), dst_pad.reshape(-1, _CHUNK)],
                   axis=1)

  ones128 = jnp.ones((_CHUNK, _H), jnp.float32)
  zeros128 = jnp.zeros((_RPT, _H), jnp.float32)

  degp = _sc_degree(idx2, ones128, zeros128)
  h1 = _matmul(x, W1)
  dinv, g1 = _post1(h1, degp[:_N, 0:1], degp[_NPAD:_NPAD + _N, 0:1])

  q = _sc_scatter(g1, idx2, src_pad, dst_pad, zeros128)
  g2 = _layer(q[:_N], q[_NPAD:_NPAD + _N], g1, dinv, b1[None, :], W2)

  q = _sc_scatter(g2, idx2, src_pad, dst_pad, zeros128)
  g3 = _layer(q[:_N], q[_NPAD:_NPAD + _N], g2, dinv, b2[None, :], W3)

  q = _sc_scatter(g3, idx2, src_pad, dst_pad, zeros128)

  wl1p = jnp.pad(Wl1, ((0, 0), (0, _H - 125)))
  bl1p = jnp.pad(bl1, (0, _H - 125))[None, :]
  wl2p = jnp.pad(Wl2, ((0, _H - 125), (0, 0)))
  wl3p = jnp.pad(Wl3, ((0, 0), (0, _H - 2)))
  bl3p = jnp.pad(bl3, (0, _H - 2))[None, :]

  out = _final(q[:_N], q[_NPAD:_NPAD + _N], g3, dinv, b3[None, :],
               wl1p, bl1p, wl2p, bl2[None, :], wl3p, bl3p, batch[:, None])
  return out[:, :2]


# speedup vs baseline: 1.2539x; 1.2539x over previous
import functools

import jax
import jax.numpy as jnp
from jax import lax
from jax.experimental import pallas as pl
from jax.experimental.pallas import tpu as pltpu
from jax.experimental.pallas import tpu_sc as plsc

_N = 10000
_E = 320000
_H = 128
_NG = 8

_NC = 2
_NS = 16
_NW = _NC * _NS
_CHUNK = 128
_EPT = 10240
_NCHUNKS = _EPT // _CHUNK
_EPAD = _EPT * _NW
_NPAD = 10240
_RPT = _NPAD // _NS

_K0 = 140
_K1 = 20

_BLK = 400
_NBLK = _N // _BLK

def _mesh():
  return plsc.VectorSubcoreMesh(core_axis_name="c", subcore_axis_name="s",
                                num_cores=_NC, num_subcores=_NS)


def _sc_degree(idx2, ones128, zeros128):
  @functools.partial(
      pl.kernel,
      mesh=_mesh(),
      out_type=jax.ShapeDtypeStruct((_NC * _NPAD, _H), jnp.float32),
      scratch_types=[
          pltpu.VMEM((2, _CHUNK), jnp.int32),
          pltpu.VMEM((2, _CHUNK), jnp.int32),
          pltpu.VMEM((_CHUNK, _H), jnp.float32),
          pltpu.VMEM_SHARED((_NPAD, _H), jnp.float32),
          pltpu.SemaphoreType.DMA,
          pltpu.SemaphoreType.DMA,
      ],
  )
  def k(idx_hbm, ones_hbm, zeros_hbm, out_hbm, i0, i1, ones_v, acc, s0, s1):
    c = lax.axis_index("c")
    s = lax.axis_index("s")
    pltpu.sync_copy(zeros_hbm, acc.at[pl.ds(s * _RPT, _RPT)])
    pltpu.sync_copy(ones_hbm, ones_v)
    plsc.subcore_barrier()
    bch = (c * _NS + s) * _NCHUNKS
    ii = (i0, i1)
    ss = (s0, s1)

    pltpu.async_copy(idx_hbm.at[bch], i0, s0).wait()
    pltpu.async_copy(idx_hbm.at[bch + 1], i1, s1)

    @pl.loop(0, _NCHUNKS // 2)
    def _(p):
      for b in range(2):
        ch = 2 * p + b
        ib, inx = ii[b], ii[1 - b]
        pltpu.sync_copy(ones_v, acc.at[ib.at[1]], add=True)

        @pl.when(ch + 2 < _NCHUNKS)
        def _():
          pltpu.async_copy(idx_hbm.at[bch + ch + 2], ib, ss[b])

        @pl.when(ch + 1 < _NCHUNKS)
        def _():
          pltpu.make_async_copy(idx_hbm.at[bch], inx, ss[1 - b]).wait()

    plsc.subcore_barrier()
    row0 = c * _NPAD + s * _RPT
    pltpu.sync_copy(acc.at[pl.ds(s * _RPT, _RPT)], out_hbm.at[pl.ds(row0, _RPT)])

  return k(idx2, ones128, zeros128)


def _sc_scatter(g, idx2, src_pad, dst_pad, zeros128):
  @functools.partial(
      pl.kernel,
      mesh=_mesh(),
      out_type=jax.ShapeDtypeStruct((_NC * _NPAD, _H), jnp.float32),
      scratch_types=[
          pltpu.VMEM((2, _CHUNK), jnp.int32),
          pltpu.VMEM((2, _CHUNK), jnp.int32),
          pltpu.VMEM((_CHUNK,), jnp.int32),
          pltpu.VMEM((_CHUNK,), jnp.int32),
          pltpu.VMEM((_CHUNK, _H), jnp.float32),
          pltpu.VMEM((_CHUNK, _H), jnp.float32),
          pltpu.VMEM_SHARED((_NPAD, _H), jnp.float32),
          pltpu.SemaphoreType.DMA,
          pltpu.SemaphoreType.DMA,
          pltpu.SemaphoreType.DMA,
          pltpu.SemaphoreType.DMA,
      ],
  )
  def k(g_hbm, idx_hbm, src_hbm, dst_hbm, zeros_hbm, out_hbm,
        i0, i1, sidx, didx, r0, r1, acc, si0, si1, sg0, sg1):
    c = lax.axis_index("c")
    s = lax.axis_index("s")
    pltpu.sync_copy(zeros_hbm, acc.at[pl.ds(s * _RPT, _RPT)])
    plsc.subcore_barrier()

    @pl.when(c == 0)
    def _():
      bch = s * _K0
      ii = (i0, i1)
      rr = (r0, r1)
      si = (si0, si1)
      sg = (sg0, sg1)
      pltpu.async_copy(idx_hbm.at[bch], i0, si0).wait()
      pltpu.async_copy(idx_hbm.at[bch + 1], i1, si1)
      pltpu.async_copy(g_hbm.at[i0.at[0]], r0, sg0)

      @pl.loop(0, (_K0 - 2) // 2)
      def _(p):
        ch0 = 2 * p
        for b in range(2):
          ib, inx = ii[b], ii[1 - b]
          rn, rx = rr[b], rr[1 - b]
          pltpu.make_async_copy(idx_hbm.at[bch], inx, si[1 - b]).wait()
          pltpu.async_copy(g_hbm.at[inx.at[0]], rx, sg[1 - b])
          pltpu.make_async_copy(g_hbm.at[ib.at[0]], rn, sg[b]).wait()
          pltpu.sync_copy(rn, acc.at[ib.at[1]], add=True)
          pltpu.async_copy(idx_hbm.at[bch + ch0 + b + 2], ib, si[b])

      pltpu.make_async_copy(idx_hbm.at[bch], i1, si1).wait()
      pltpu.async_copy(g_hbm.at[i1.at[0]], r1, sg1)
      pltpu.make_async_copy(g_hbm.at[i0.at[0]], r0, sg0).wait()
      pltpu.sync_copy(r0, acc.at[i0.at[1]], add=True)
      pltpu.make_async_copy(g_hbm.at[i1.at[0]], r1, sg1).wait()
      pltpu.sync_copy(r1, acc.at[i1.at[1]], add=True)

    if _K1:
      @pl.when(c == 1)
      def _():
        base = (_NS * _K0 + s * _K1) * _CHUNK

        @pl.loop(0, _K1)
        def _(ch):
          off = base + ch * _CHUNK
          pltpu.sync_copy(src_hbm.at[pl.ds(off, _CHUNK)], sidx)
          pltpu.sync_copy(dst_hbm.at[pl.ds(off, _CHUNK)], didx)
          pltpu.async_copy(g_hbm.at[sidx], r0, sg0).wait()
          pltpu.sync_copy(r0, acc.at[didx], add=True)

    plsc.subcore_barrier()
    row0 = c * _NPAD + s * _RPT
    pltpu.sync_copy(acc.at[pl.ds(s * _RPT, _RPT)], out_hbm.at[pl.ds(row0, _RPT)])

  return k(g, idx2, src_pad, dst_pad, zeros128)


def _mm_body(x_ref, w_ref, o_ref):
  o_ref[...] = jnp.dot(x_ref[...], w_ref[...], preferred_element_type=jnp.float32)


def _matmul(x, w):
  return pl.pallas_call(
      _mm_body,
      grid=(_NBLK,),
      in_specs=[
          pl.BlockSpec((_BLK, x.shape[1]), lambda i: (i, 0)),
          pl.BlockSpec(w.shape, lambda i: (0, 0)),
      ],
      out_specs=pl.BlockSpec((_BLK, w.shape[1]), lambda i: (i, 0)),
      out_shape=jax.ShapeDtypeStruct((_N, w.shape[1]), jnp.float32),
      compiler_params=pltpu.CompilerParams(dimension_semantics=("parallel",)),
  )(x, w)


def _post1_body(h_ref, p0_ref, p1_ref, dinv_ref, g_ref):
  deg = p0_ref[...] + p1_ref[...] + 1.0
  dinv = lax.rsqrt(deg)
  dinv_ref[...] = dinv
  g_ref[...] = h_ref[...] * dinv


def _post1(h1, p0, p1):
  return pl.pallas_call(
      _post1_body,
      grid=(_NBLK,),
      in_specs=[
          pl.BlockSpec((_BLK, _H), lambda i: (i, 0)),
          pl.BlockSpec((_BLK, 1), lambda i: (i, 0)),
          pl.BlockSpec((_BLK, 1), lambda i: (i, 0)),
      ],
      out_specs=[
          pl.BlockSpec((_BLK, 1), lambda i: (i, 0)),
          pl.BlockSpec((_BLK, _H), lambda i: (i, 0)),
      ],
      out_shape=[
          jax.ShapeDtypeStruct((_N, 1), jnp.float32),
          jax.ShapeDtypeStruct((_N, _H), jnp.float32),
      ],
      compiler_params=pltpu.CompilerParams(dimension_semantics=("parallel",)),
  )(h1, p0, p1)


def _layer_body(p0_ref, p1_ref, g_ref, dinv_ref, b_ref, w_ref, o_ref):
  dinv = dinv_ref[...]
  t = dinv * (p0_ref[...] + p1_ref[...] + g_ref[...]) + b_ref[...]
  z = jnp.maximum(t, 0.0)
  o_ref[...] = dinv * jnp.dot(z, w_ref[...], preferred_element_type=jnp.float32)


def _layer(p0, p1, g, dinv, b, w):
  return pl.pallas_call(
      _layer_body,
      grid=(_NBLK,),
      in_specs=[
          pl.BlockSpec((_BLK, _H), lambda i: (i, 0)),
          pl.BlockSpec((_BLK, _H), lambda i: (i, 0)),
          pl.BlockSpec((_BLK, _H), lambda i: (i, 0)),
          pl.BlockSpec((_BLK, 1), lambda i: (i, 0)),
          pl.BlockSpec((1, _H), lambda i: (0, 0)),
          pl.BlockSpec((_H, _H), lambda i: (0, 0)),
      ],
      out_specs=pl.BlockSpec((_BLK, _H), lambda i: (i, 0)),
      out_shape=jax.ShapeDtypeStruct((_N, _H), jnp.float32),
      compiler_params=pltpu.CompilerParams(dimension_semantics=("parallel",)),
  )(p0, p1, g, dinv, b, w)


def _final_body(p0_ref, p1_ref, g_ref, dinv_ref, b3_ref, wl1_ref, bl1_ref,
                wl2_ref, bl2_ref, wl3_ref, bl3_ref, batch_ref, out_ref,
                cnt_ref):
  i = pl.program_id(0)

  @pl.when(i == 0)
  def _():
    out_ref[...] = jnp.zeros_like(out_ref)
    cnt_ref[...] = jnp.zeros_like(cnt_ref)

  dinv = dinv_ref[...]
  t = dinv * (p0_ref[...] + p1_ref[...] + g_ref[...]) + b3_ref[...]
  h1m = jnp.maximum(
      jnp.dot(t, wl1_ref[...], preferred_element_type=jnp.float32)
      + bl1_ref[...], 0.0)
  hm = jnp.maximum(
      jnp.dot(h1m, wl2_ref[...], preferred_element_type=jnp.float32)
      + bl2_ref[...] + t, 0.0)
  hf = jnp.maximum(
      jnp.dot(hm, wl3_ref[...], preferred_element_type=jnp.float32)
      + bl3_ref[...], 0.0)
  onehot = (batch_ref[...] ==
            lax.broadcasted_iota(jnp.int32, (1, _NG), 1)).astype(jnp.float32)
  out_ref[...] += lax.dot_general(
      onehot, hf, (((0,), (0,)), ((), ())), preferred_element_type=jnp.float32)
  cnt_ref[...] += jnp.broadcast_to(jnp.sum(onehot, axis=0)[:, None],
                                   (_NG, _H))

  @pl.when(i == _NBLK - 1)
  def _():
    out_ref[...] = out_ref[...] / jnp.maximum(cnt_ref[...], 1.0)


def _final(p0, p1, g, dinv, b3, wl1, bl1, wl2, bl2, wl3, bl3, batch2d):
  row = lambda i: (i, 0)
  fixed = lambda i: (0, 0)
  return pl.pallas_call(
      _final_body,
      grid=(_NBLK,),
      in_specs=[
          pl.BlockSpec((_BLK, _H), row),
          pl.BlockSpec((_BLK, _H), row),
          pl.BlockSpec((_BLK, _H), row),
          pl.BlockSpec((_BLK, 1), row),
          pl.BlockSpec((1, _H), fixed),
          pl.BlockSpec((_H, _H), fixed),
          pl.BlockSpec((1, _H), fixed),
          pl.BlockSpec((_H, _H), fixed),
          pl.BlockSpec((1, _H), fixed),
          pl.BlockSpec((_H, _H), fixed),
          pl.BlockSpec((1, _H), fixed),
          pl.BlockSpec((_BLK, 1), row),
      ],
      out_specs=pl.BlockSpec((_NG, _H), fixed),
      out_shape=jax.ShapeDtypeStruct((_NG, _H), jnp.float32),
      scratch_shapes=[pltpu.VMEM((_NG, _H), jnp.float32)],
      compiler_params=pltpu.CompilerParams(dimension_semantics=("arbitrary",)),
  )(p0, p1, g, dinv, b3, wl1, bl1, wl2, bl2, wl3, bl3, batch2d)


def kernel(x, edge_index, batch, pos, W1, b1, W2, b2, W3, b3,
           Wl1, bl1, Wl2, bl2, Wl3, bl3):
  del pos
  src = edge_index[0]
  dst = edge_index[1]
  npad = _EPAD - _E
  src_pad = jnp.concatenate([src, jnp.zeros((npad,), jnp.int32)])
  dst_pad = jnp.concatenate([dst, jnp.full((npad,), _N, jnp.int32)])

  idx2 = jnp.stack([src_pad.reshape(-1, _CHUNK), dst_pad.reshape(-1, _CHUNK)],
                   axis=1)

  ones128 = jnp.ones((_CHUNK, _H), jnp.float32)
  zeros128 = jnp.zeros((_RPT, _H), jnp.float32)

  degp = _sc_degree(idx2, ones128, zeros128)
  h1 = _matmul(x, W1)
  dinv, g1 = _post1(h1, degp[:_N, 0:1], degp[_NPAD:_NPAD + _N, 0:1])

  q = _sc_scatter(g1, idx2, src_pad, dst_pad, zeros128)
  g2 = _layer(q[:_N], q[_NPAD:_NPAD + _N], g1, dinv, b1[None, :], W2)

  q = _sc_scatter(g2, idx2, src_pad, dst_pad, zeros128)
  g3 = _layer(q[:_N], q[_NPAD:_NPAD + _N], g2, dinv, b2[None, :], W3)

  q = _sc_scatter(g3, idx2, src_pad, dst_pad, zeros128)

  wl1p = jnp.pad(Wl1, ((0, 0), (0, _H - 125)))
  bl1p = jnp.pad(bl1, (0, _H - 125))[None, :]
  wl2p = jnp.pad(Wl2, ((0, _H - 125), (0, 0)))
  wl3p = jnp.pad(Wl3, ((0, 0), (0, _H - 2)))
  bl3p = jnp.pad(bl3, (0, _H - 2))[None, :]

  out = _final(q[:_N], q[_NPAD:_NPAD + _N], g3, dinv, b3[None, :],
               wl1p, bl1p, wl2p, bl2[None, :], wl3p, bl3p, batch[:, None])
  return out[:, :2]


# speedup vs baseline: 1.2815x; 1.0220x over previous
import functools

import jax
import jax.numpy as jnp
from jax import lax
from jax.experimental import pallas as pl
from jax.experimental.pallas import tpu as pltpu
from jax.experimental.pallas import tpu_sc as plsc

_N = 10000
_E = 320000
_H = 128
_NG = 8

_NC = 2
_NS = 16
_NW = _NC * _NS
_CHUNK = 128
_EPT = 10240
_NCHUNKS = _EPT // _CHUNK
_EPAD = _EPT * _NW
_NPAD = 10240
_RPT = _NPAD // _NS

_K0 = 148
_K1 = 12

_BLK = 400
_NBLK = _N // _BLK

def _mesh():
  return plsc.VectorSubcoreMesh(core_axis_name="c", subcore_axis_name="s",
                                num_cores=_NC, num_subcores=_NS)


def _sc_degree(idx2, ones128, zeros128):
  @functools.partial(
      pl.kernel,
      mesh=_mesh(),
      out_type=jax.ShapeDtypeStruct((_NC * _NPAD, _H), jnp.float32),
      scratch_types=[
          pltpu.VMEM((2, _CHUNK), jnp.int32),
          pltpu.VMEM((2, _CHUNK), jnp.int32),
          pltpu.VMEM((_CHUNK, _H), jnp.float32),
          pltpu.VMEM_SHARED((_NPAD, _H), jnp.float32),
          pltpu.SemaphoreType.DMA,
          pltpu.SemaphoreType.DMA,
      ],
  )
  def k(idx_hbm, ones_hbm, zeros_hbm, out_hbm, i0, i1, ones_v, acc, s0, s1):
    c = lax.axis_index("c")
    s = lax.axis_index("s")
    pltpu.sync_copy(zeros_hbm, acc.at[pl.ds(s * _RPT, _RPT)])
    pltpu.sync_copy(ones_hbm, ones_v)
    plsc.subcore_barrier()
    bch = (c * _NS + s) * _NCHUNKS
    ii = (i0, i1)
    ss = (s0, s1)

    pltpu.async_copy(idx_hbm.at[bch], i0, s0).wait()
    pltpu.async_copy(idx_hbm.at[bch + 1], i1, s1)

    @pl.loop(0, _NCHUNKS // 2)
    def _(p):
      for b in range(2):
        ch = 2 * p + b
        ib, inx = ii[b], ii[1 - b]
        pltpu.sync_copy(ones_v, acc.at[ib.at[1]], add=True)

        @pl.when(ch + 2 < _NCHUNKS)
        def _():
          pltpu.async_copy(idx_hbm.at[bch + ch + 2], ib, ss[b])

        @pl.when(ch + 1 < _NCHUNKS)
        def _():
          pltpu.make_async_copy(idx_hbm.at[bch], inx, ss[1 - b]).wait()

    plsc.subcore_barrier()
    row0 = c * _NPAD + s * _RPT
    pltpu.sync_copy(acc.at[pl.ds(s * _RPT, _RPT)], out_hbm.at[pl.ds(row0, _RPT)])

  return k(idx2, ones128, zeros128)


def _sc_scatter(g, idx2, src_pad, dst_pad, zeros128):
  @functools.partial(
      pl.kernel,
      mesh=_mesh(),
      out_type=jax.ShapeDtypeStruct((_NC * _NPAD, _H), jnp.float32),
      scratch_types=[
          pltpu.VMEM((2, _CHUNK), jnp.int32),
          pltpu.VMEM((2, _CHUNK), jnp.int32),
          pltpu.VMEM((_CHUNK,), jnp.int32),
          pltpu.VMEM((_CHUNK,), jnp.int32),
          pltpu.VMEM((_CHUNK, _H), jnp.float32),
          pltpu.VMEM((_CHUNK, _H), jnp.float32),
          pltpu.VMEM_SHARED((_NPAD, _H), jnp.float32),
          pltpu.SemaphoreType.DMA,
          pltpu.SemaphoreType.DMA,
          pltpu.SemaphoreType.DMA,
          pltpu.SemaphoreType.DMA,
      ],
  )
  def k(g_hbm, idx_hbm, src_hbm, dst_hbm, zeros_hbm, out_hbm,
        i0, i1, sidx, didx, r0, r1, acc, si0, si1, sg0, sg1):
    c = lax.axis_index("c")
    s = lax.axis_index("s")
    pltpu.sync_copy(zeros_hbm, acc.at[pl.ds(s * _RPT, _RPT)])
    plsc.subcore_barrier()

    @pl.when(c == 0)
    def _():
      bch = s * _K0
      ii = (i0, i1)
      rr = (r0, r1)
      si = (si0, si1)
      sg = (sg0, sg1)
      pltpu.async_copy(idx_hbm.at[bch], i0, si0).wait()
      pltpu.async_copy(idx_hbm.at[bch + 1], i1, si1)
      pltpu.async_copy(g_hbm.at[i0.at[0]], r0, sg0)

      @pl.loop(0, (_K0 - 2) // 2)
      def _(p):
        ch0 = 2 * p
        for b in range(2):
          ib, inx = ii[b], ii[1 - b]
          rn, rx = rr[b], rr[1 - b]
          pltpu.make_async_copy(idx_hbm.at[bch], inx, si[1 - b]).wait()
          pltpu.async_copy(g_hbm.at[inx.at[0]], rx, sg[1 - b])
          pltpu.make_async_copy(g_hbm.at[ib.at[0]], rn, sg[b]).wait()
          pltpu.sync_copy(rn, acc.at[ib.at[1]], add=True)
          pltpu.async_copy(idx_hbm.at[bch + ch0 + b + 2], ib, si[b])

      pltpu.make_async_copy(idx_hbm.at[bch], i1, si1).wait()
      pltpu.async_copy(g_hbm.at[i1.at[0]], r1, sg1)
      pltpu.make_async_copy(g_hbm.at[i0.at[0]], r0, sg0).wait()
      pltpu.sync_copy(r0, acc.at[i0.at[1]], add=True)
      pltpu.make_async_copy(g_hbm.at[i1.at[0]], r1, sg1).wait()
      pltpu.sync_copy(r1, acc.at[i1.at[1]], add=True)

    if _K1:
      @pl.when(c == 1)
      def _():
        base = (_NS * _K0 + s * _K1) * _CHUNK

        @pl.loop(0, _K1)
        def _(ch):
          off = base + ch * _CHUNK
          pltpu.sync_copy(src_hbm.at[pl.ds(off, _CHUNK)], sidx)
          pltpu.sync_copy(dst_hbm.at[pl.ds(off, _CHUNK)], didx)
          pltpu.async_copy(g_hbm.at[sidx], r0, sg0).wait()
          pltpu.sync_copy(r0, acc.at[didx], add=True)

    plsc.subcore_barrier()
    row0 = c * _NPAD + s * _RPT
    pltpu.sync_copy(acc.at[pl.ds(s * _RPT, _RPT)], out_hbm.at[pl.ds(row0, _RPT)])

  return k(g, idx2, src_pad, dst_pad, zeros128)


def _mm_body(x_ref, w_ref, o_ref):
  o_ref[...] = jnp.dot(x_ref[...], w_ref[...], preferred_element_type=jnp.float32)


def _matmul(x, w):
  return pl.pallas_call(
      _mm_body,
      grid=(_NBLK,),
      in_specs=[
          pl.BlockSpec((_BLK, x.shape[1]), lambda i: (i, 0)),
          pl.BlockSpec(w.shape, lambda i: (0, 0)),
      ],
      out_specs=pl.BlockSpec((_BLK, w.shape[1]), lambda i: (i, 0)),
      out_shape=jax.ShapeDtypeStruct((_N, w.shape[1]), jnp.float32),
      compiler_params=pltpu.CompilerParams(dimension_semantics=("parallel",)),
  )(x, w)


def _post1_body(h_ref, p0_ref, p1_ref, dinv_ref, g_ref):
  deg = p0_ref[...] + p1_ref[...] + 1.0
  dinv = lax.rsqrt(deg)
  dinv_ref[...] = dinv
  g_ref[...] = h_ref[...] * dinv


def _post1(h1, p0, p1):
  return pl.pallas_call(
      _post1_body,
      grid=(_NBLK,),
      in_specs=[
          pl.BlockSpec((_BLK, _H), lambda i: (i, 0)),
          pl.BlockSpec((_BLK, 1), lambda i: (i, 0)),
          pl.BlockSpec((_BLK, 1), lambda i: (i, 0)),
      ],
      out_specs=[
          pl.BlockSpec((_BLK, 1), lambda i: (i, 0)),
          pl.BlockSpec((_BLK, _H), lambda i: (i, 0)),
      ],
      out_shape=[
          jax.ShapeDtypeStruct((_N, 1), jnp.float32),
          jax.ShapeDtypeStruct((_N, _H), jnp.float32),
      ],
      compiler_params=pltpu.CompilerParams(dimension_semantics=("parallel",)),
  )(h1, p0, p1)


def _layer_body(p0_ref, p1_ref, g_ref, dinv_ref, b_ref, w_ref, o_ref):
  dinv = dinv_ref[...]
  t = dinv * (p0_ref[...] + p1_ref[...] + g_ref[...]) + b_ref[...]
  z = jnp.maximum(t, 0.0)
  o_ref[...] = dinv * jnp.dot(z, w_ref[...], preferred_element_type=jnp.float32)


def _layer(p0, p1, g, dinv, b, w):
  return pl.pallas_call(
      _layer_body,
      grid=(_NBLK,),
      in_specs=[
          pl.BlockSpec((_BLK, _H), lambda i: (i, 0)),
          pl.BlockSpec((_BLK, _H), lambda i: (i, 0)),
          pl.BlockSpec((_BLK, _H), lambda i: (i, 0)),
          pl.BlockSpec((_BLK, 1), lambda i: (i, 0)),
          pl.BlockSpec((1, _H), lambda i: (0, 0)),
          pl.BlockSpec((_H, _H), lambda i: (0, 0)),
      ],
      out_specs=pl.BlockSpec((_BLK, _H), lambda i: (i, 0)),
      out_shape=jax.ShapeDtypeStruct((_N, _H), jnp.float32),
      compiler_params=pltpu.CompilerParams(dimension_semantics=("parallel",)),
  )(p0, p1, g, dinv, b, w)


def _final_body(p0_ref, p1_ref, g_ref, dinv_ref, b3_ref, wl1_ref, bl1_ref,
                wl2_ref, bl2_ref, wl3_ref, bl3_ref, batch_ref, out_ref,
                cnt_ref):
  i = pl.program_id(0)

  @pl.when(i == 0)
  def _():
    out_ref[...] = jnp.zeros_like(out_ref)
    cnt_ref[...] = jnp.zeros_like(cnt_ref)

  dinv = dinv_ref[...]
  t = dinv * (p0_ref[...] + p1_ref[...] + g_ref[...]) + b3_ref[...]
  h1m = jnp.maximum(
      jnp.dot(t, wl1_ref[...], preferred_element_type=jnp.float32)
      + bl1_ref[...], 0.0)
  hm = jnp.maximum(
      jnp.dot(h1m, wl2_ref[...], preferred_element_type=jnp.float32)
      + bl2_ref[...] + t, 0.0)
  hf = jnp.maximum(
      jnp.dot(hm, wl3_ref[...], preferred_element_type=jnp.float32)
      + bl3_ref[...], 0.0)
  onehot = (batch_ref[...] ==
            lax.broadcasted_iota(jnp.int32, (1, _NG), 1)).astype(jnp.float32)
  out_ref[...] += lax.dot_general(
      onehot, hf, (((0,), (0,)), ((), ())), preferred_element_type=jnp.float32)
  cnt_ref[...] += jnp.broadcast_to(jnp.sum(onehot, axis=0)[:, None],
                                   (_NG, _H))

  @pl.when(i == _NBLK - 1)
  def _():
    out_ref[...] = out_ref[...] / jnp.maximum(cnt_ref[...], 1.0)


def _final(p0, p1, g, dinv, b3, wl1, bl1, wl2, bl2, wl3, bl3, batch2d):
  row = lambda i: (i, 0)
  fixed = lambda i: (0, 0)
  return pl.pallas_call(
      _final_body,
      grid=(_NBLK,),
      in_specs=[
          pl.BlockSpec((_BLK, _H), row),
          pl.BlockSpec((_BLK, _H), row),
          pl.BlockSpec((_BLK, _H), row),
          pl.BlockSpec((_BLK, 1), row),
          pl.BlockSpec((1, _H), fixed),
          pl.BlockSpec((_H, _H), fixed),
          pl.BlockSpec((1, _H), fixed),
          pl.BlockSpec((_H, _H), fixed),
          pl.BlockSpec((1, _H), fixed),
          pl.BlockSpec((_H, _H), fixed),
          pl.BlockSpec((1, _H), fixed),
          pl.BlockSpec((_BLK, 1), row),
      ],
      out_specs=pl.BlockSpec((_NG, _H), fixed),
      out_shape=jax.ShapeDtypeStruct((_NG, _H), jnp.float32),
      scratch_shapes=[pltpu.VMEM((_NG, _H), jnp.float32)],
      compiler_params=pltpu.CompilerParams(dimension_semantics=("arbitrary",)),
  )(p0, p1, g, dinv, b3, wl1, bl1, wl2, bl2, wl3, bl3, batch2d)


def kernel(x, edge_index, batch, pos, W1, b1, W2, b2, W3, b3,
           Wl1, bl1, Wl2, bl2, Wl3, bl3):
  del pos
  src = edge_index[0]
  dst = edge_index[1]
  npad = _EPAD - _E
  src_pad = jnp.concatenate([src, jnp.zeros((npad,), jnp.int32)])
  dst_pad = jnp.concatenate([dst, jnp.full((npad,), _N, jnp.int32)])

  idx2 = jnp.stack([src_pad.reshape(-1, _CHUNK), dst_pad.reshape(-1, _CHUNK)],
                   axis=1)

  ones128 = jnp.ones((_CHUNK, _H), jnp.float32)
  zeros128 = jnp.zeros((_RPT, _H), jnp.float32)

  degp = _sc_degree(idx2, ones128, zeros128)
  h1 = _matmul(x, W1)
  dinv, g1 = _post1(h1, degp[:_N, 0:1], degp[_NPAD:_NPAD + _N, 0:1])

  q = _sc_scatter(g1, idx2, src_pad, dst_pad, zeros128)
  g2 = _layer(q[:_N], q[_NPAD:_NPAD + _N], g1, dinv, b1[None, :], W2)

  q = _sc_scatter(g2, idx2, src_pad, dst_pad, zeros128)
  g3 = _layer(q[:_N], q[_NPAD:_NPAD + _N], g2, dinv, b2[None, :], W3)

  q = _sc_scatter(g3, idx2, src_pad, dst_pad, zeros128)

  wl1p = jnp.pad(Wl1, ((0, 0), (0, _H - 125)))
  bl1p = jnp.pad(bl1, (0, _H - 125))[None, :]
  wl2p = jnp.pad(Wl2, ((0, _H - 125), (0, 0)))
  wl3p = jnp.pad(Wl3, ((0, 0), (0, _H - 2)))
  bl3p = jnp.pad(bl3, (0, _H - 2))[None, :]

  out = _final(q[:_N], q[_NPAD:_NPAD + _N], g3, dinv, b3[None, :],
               wl1p, bl1p, wl2p, bl2[None, :], wl3p, bl3p, batch[:, None])
  return out[:, :2]
